# baseline (speedup 1.0000x reference)
import functools

import jax
import jax.numpy as jnp
from jax import lax
from jax.experimental import pallas as pl
from jax.experimental.pallas import tpu as pltpu
from jax.experimental.pallas import tpu_sc as plsc

NBLK = 256
SC_K = 128
SC_NC = 1
SC_NS = 16


def _ceil_to(x, m):
    return (x + m - 1) // m * m


def _prep_body(wc_ref, wb_ref, a3_ref, a4_ref, ab_ref, attn_ref, w_ref, t_ref):
    w_ref[0] = jnp.dot(wc_ref[0], wb_ref[0], preferred_element_type=jnp.float32)
    pet = jnp.dot(attn_ref[...], a3_ref[0], preferred_element_type=jnp.float32)
    pel = jnp.dot(attn_ref[...], a4_ref[0], preferred_element_type=jnp.float32)
    t = pet[:, None, :] + pel[None, :, :] + ab_ref[...]
    t_ref[0] = t


def _prep_tables(w_comp, bases_flat, a3, a4, a_b, attn_tab, L, R, NB, D, AD):
    return pl.pallas_call(
        _prep_body,
        grid=(L,),
        in_specs=[
            pl.BlockSpec((1, R, NB), lambda l: (l, 0, 0)),
            pl.BlockSpec((1, NB, D * D), lambda l: (l, 0, 0)),
            pl.BlockSpec((1, AD, D), lambda l: (l, 0, 0)),
            pl.BlockSpec((1, AD, D), lambda l: (l, 0, 0)),
            pl.BlockSpec((1, 1, D), lambda l: (l, 0, 0)),
            pl.BlockSpec((R, AD), lambda l: (0, 0)),
        ],
        out_specs=[
            pl.BlockSpec((1, R, D * D), lambda l: (l, 0, 0)),
            pl.BlockSpec((1, R, R, D), lambda l: (l, 0, 0, 0)),
        ],
        out_shape=[
            jax.ShapeDtypeStruct((L, R, D * D), jnp.float32),
            jax.ShapeDtypeStruct((L, R, R, D), jnp.float32),
        ],
    )(w_comp, bases_flat, a3, a4, a_b, attn_tab)


def _proj_body(h_ref, w_ref, a1_ref, a2_ref, xt_ref, ps_ref, pd_ref, *, R):
    hb = h_ref[...]
    for r in range(R):
        xt_ref[r] = jnp.dot(hb, w_ref[r], preferred_element_type=jnp.float32)
    ps_ref[...] = jnp.dot(hb, a1_ref[...], preferred_element_type=jnp.float32)
    pd_ref[...] = jnp.dot(hb, a2_ref[...], preferred_element_type=jnp.float32)


def _layer_proj(h, w_l, a1, a2, R, D, npad):
    nb = npad // NBLK
    return pl.pallas_call(
        functools.partial(_proj_body, R=R),
        grid=(nb,),
        in_specs=[
            pl.BlockSpec((NBLK, D), lambda i: (i, 0)),
            pl.BlockSpec((R, D, D), lambda i: (0, 0, 0)),
            pl.BlockSpec((D, D), lambda i: (0, 0)),
            pl.BlockSpec((D, D), lambda i: (0, 0)),
        ],
        out_specs=[
            pl.BlockSpec((R, NBLK, D), lambda i: (0, i, 0)),
            pl.BlockSpec((NBLK, D), lambda i: (i, 0)),
            pl.BlockSpec((NBLK, D), lambda i: (i, 0)),
        ],
        out_shape=[
            jax.ShapeDtypeStruct((R, npad, D), jnp.float32),
            jax.ShapeDtypeStruct((npad, D), jnp.float32),
            jax.ShapeDtypeStruct((npad, D), jnp.float32),
        ],
    )(h, w_l, a1, a2)


def _update_body(h_ref, sl_ref, agg_ref, out_ref):
    cur = jnp.dot(h_ref[...], sl_ref[...], preferred_element_type=jnp.float32)
    out_ref[...] = jnp.maximum(cur + agg_ref[...], 0.0)


def _layer_update(h, sl_l, agg, D, npad):
    nb = npad // NBLK
    return pl.pallas_call(
        _update_body,
        grid=(nb,),
        in_specs=[
            pl.BlockSpec((NBLK, D), lambda i: (i, 0)),
            pl.BlockSpec((D, D), lambda i: (0, 0)),
            pl.BlockSpec((NBLK, D), lambda i: (i, 0)),
        ],
        out_specs=pl.BlockSpec((NBLK, D), lambda i: (i, 0)),
        out_shape=jax.ShapeDtypeStruct((npad, D), jnp.float32),
    )(h, sl_l, agg)


def _readout_body(rep_ref, ids_ref, rtab_ref, fcw_ref, fcb_ref,
                  head_ref, tail_ref, rl_ref, out_ref, *, nb, B, LD, RD):
    def blk(i, carry):
        acc, cnt = carry
        idsb = ids_ref[pl.ds(i, 1), :]
        oh = (lax.broadcasted_iota(jnp.int32, (B, NBLK), 0) == idsb)
        ohf = oh.astype(jnp.float32)
        chunk = rep_ref[pl.ds(i * NBLK, NBLK), :]
        acc = acc + jnp.dot(ohf, chunk, preferred_element_type=jnp.float32)
        cnt = cnt + jnp.sum(ohf, axis=1, keepdims=True)
        return acc, cnt

    acc0 = jnp.zeros((B, LD), jnp.float32)
    cnt0 = jnp.zeros((B, 1), jnp.float32)
    acc, cnt = lax.fori_loop(0, nb, blk, (acc0, cnt0))
    g_out = acc / jnp.maximum(cnt, 1.0)
    heads = jnp.concatenate(
        [rep_ref[pl.ds(head_ref[b], 1), :] for b in range(B)], axis=0)
    tails = jnp.concatenate(
        [rep_ref[pl.ds(tail_ref[b], 1), :] for b in range(B)], axis=0)
    rels = jnp.concatenate(
        [rtab_ref[pl.ds(rl_ref[b], 1), :] for b in range(B)], axis=0)
    g_rep = jnp.concatenate([g_out, heads, tails, rels], axis=1)
    out_ref[...] = (jnp.dot(g_rep, fcw_ref[...],
                            preferred_element_type=jnp.float32)
                    + fcb_ref[0, 0])


def _readout(rep_flat, ids2d, rel_tab, fc_w, fc_b2, head_ids, tail_ids,
             rel_labels, B, LD, RD, npad):
    nb = npad // NBLK
    return pl.pallas_call(
        functools.partial(_readout_body, nb=nb, B=B, LD=LD, RD=RD),
        in_specs=[
            pl.BlockSpec((npad, LD), lambda: (0, 0)),
            pl.BlockSpec((nb, NBLK), lambda: (0, 0)),
            pl.BlockSpec((B, RD), lambda: (0, 0)),
            pl.BlockSpec((LD * 3 + RD, 1), lambda: (0, 0)),
            pl.BlockSpec((1, 1), lambda: (0, 0)),
            pl.BlockSpec(memory_space=pltpu.SMEM),
            pl.BlockSpec(memory_space=pltpu.SMEM),
            pl.BlockSpec(memory_space=pltpu.SMEM),
        ],
        out_specs=pl.BlockSpec((B, 1), lambda: (0, 0)),
        out_shape=jax.ShapeDtypeStruct((B, 1), jnp.float32),
    )(rep_flat, ids2d, rel_tab, fc_w, fc_b2, head_ids, tail_ids, rel_labels)


def _make_attn_kernel(epad, D):
    nworkers = 2 * SC_NS
    pw = epad // nworkers
    nchunks = pw // SC_K
    assert nchunks % 2 == 0
    mesh = plsc.VectorSubcoreMesh(core_axis_name="c", subcore_axis_name="s",
                                  num_cores=2)

    @functools.partial(
        pl.kernel,
        out_type=jax.ShapeDtypeStruct((epad,), jnp.float32),
        mesh=mesh,
        compiler_params=pltpu.CompilerParams(needs_layout_passes=False),
        scratch_types=[
            pltpu.VMEM((pw,), jnp.int32),
            pltpu.VMEM((pw,), jnp.int32),
            pltpu.VMEM((pw,), jnp.int32),
            pltpu.VMEM((2, SC_K, D), jnp.float32),
            pltpu.VMEM((2, SC_K, D), jnp.float32),
            pltpu.VMEM((2, SC_K, D), jnp.float32),
            pltpu.VMEM((2, SC_K), jnp.float32),
            pltpu.VMEM((D,), jnp.float32),
            pltpu.VMEM((16,), jnp.float32),
            pltpu.VMEM((256,), jnp.float32),
            pltpu.SemaphoreType.DMA,
            pltpu.SemaphoreType.DMA,
        ],
    )
    def attn_kernel(ps_hbm, pd_hbm, t_hbm, isrc_hbm, idst_hbm, it_hbm,
                    bw_hbm, bb_hbm, a_hbm,
                    isrc_all, idst_all, it_all, g_ps, g_pd, g_t, a_buf,
                    bw_v, bb_v, acc_buf, sem0, sem1):
        cid = lax.axis_index("c")
        sid = lax.axis_index("s")
        wid = sid * 2 + cid
        tbase = wid * pw
        sems = (sem0, sem1)

        pc = pw // 8
        for q in range(8):
            pltpu.sync_copy(isrc_hbm.at[pl.ds(tbase + q * pc, pc)],
                            isrc_all.at[pl.ds(q * pc, pc)])
            pltpu.sync_copy(idst_hbm.at[pl.ds(tbase + q * pc, pc)],
                            idst_all.at[pl.ds(q * pc, pc)])
            pltpu.sync_copy(it_hbm.at[pl.ds(tbase + q * pc, pc)],
                            it_all.at[pl.ds(q * pc, pc)])
        pltpu.sync_copy(bw_hbm, bw_v)
        pltpu.sync_copy(bb_hbm, bb_v)
        bw_regs = [bw_v[pl.ds(s * 16, 16)] for s in range(D // 16)]
        bb16 = bb_v[...]
        lane16 = lax.broadcasted_iota(jnp.int32, (16,), 0) * 16

        def issue(j, b):
            sl = pl.ds(j * SC_K, SC_K)
            pltpu.async_copy(ps_hbm.at[isrc_all.at[sl]], g_ps.at[b], sems[b])
            pltpu.async_copy(pd_hbm.at[idst_all.at[sl]], g_pd.at[b], sems[b])
            pltpu.async_copy(t_hbm.at[it_all.at[sl]], g_t.at[b], sems[b])

        def drain(j, b):
            sl = pl.ds(j * SC_K, SC_K)
            pltpu.make_async_copy(
                ps_hbm.at[isrc_all.at[sl]], g_ps.at[b], sems[b]).wait()
            pltpu.make_async_copy(
                pd_hbm.at[idst_all.at[sl]], g_pd.at[b], sems[b]).wait()
            pltpu.make_async_copy(
                t_hbm.at[it_all.at[sl]], g_t.at[b], sems[b]).wait()

        issue(0, 0)
        issue(1, 1)

        def body(jj, _):
            for b in range(2):
                j = jj * 2 + b
                drain(j, b)
                psb, pdb, tb = g_ps.at[b], g_pd.at[b], g_t.at[b]

                def group(g, _):
                    e0 = g * 16
                    for ee in range(16):
                        e = e0 + ee
                        acc = bb16
                        for s in range(D // 16):
                            sl = pl.ds(s * 16, 16)
                            pre = psb[e, sl] + pdb[e, sl] + tb[e, sl]
                            acc = acc + jnp.maximum(pre, 0.0) * bw_regs[s]
                        acc_buf[pl.ds(ee * 16, 16)] = acc
                    tots = jnp.zeros((16,), jnp.float32)
                    for s in range(16):
                        tots = tots + plsc.load_gather(acc_buf, [lane16 + s])
                    av = 1.0 / (1.0 + jnp.exp(-tots))
                    a_buf.at[b][pl.ds(e0, 16)] = av
                    return 0
                lax.fori_loop(0, SC_K // 16, group, 0)
                @pl.when(j + 2 < nchunks)
                def _():
                    issue_j = j + 2
                    sl = pl.ds(issue_j * SC_K, SC_K)
                    pltpu.async_copy(ps_hbm.at[isrc_all.at[sl]],
                                     g_ps.at[b], sems[b])
                    pltpu.async_copy(pd_hbm.at[idst_all.at[sl]],
                                     g_pd.at[b], sems[b])
                    pltpu.async_copy(t_hbm.at[it_all.at[sl]],
                                     g_t.at[b], sems[b])
                pltpu.sync_copy(a_buf.at[b],
                                a_hbm.at[pl.ds(tbase + j * SC_K, SC_K)])
            return 0

        lax.fori_loop(0, nchunks // 2, body, 0)

    return attn_kernel


def _make_scat_kernel(npad, epad, D, rng):
    half = rng
    TRASH = half
    r_agg = half + 128
    nworkers = SC_NS
    pw = epad // nworkers
    nchunks = pw // SC_K
    assert nchunks % 2 == 0
    zrows = r_agg // SC_NS
    drows = half // SC_NS
    mesh = plsc.VectorSubcoreMesh(core_axis_name="c", subcore_axis_name="s",
                                  num_cores=2)

    @functools.partial(
        pl.kernel,
        out_type=jax.ShapeDtypeStruct((2, half, D), jnp.float32),
        mesh=mesh,
        compiler_params=pltpu.CompilerParams(needs_layout_passes=False),
        scratch_types=[
            pltpu.VMEM((pw,), jnp.int32),
            pltpu.VMEM((pw,), jnp.int32),
            pltpu.VMEM((pw,), jnp.float32),
            pltpu.VMEM((2, SC_K), jnp.int32),
            pltpu.VMEM((2, SC_K, D), jnp.float32),
            pltpu.VMEM((2, SC_K, D), jnp.float32),
            pltpu.VMEM((16,), jnp.int32),
            pltpu.VMEM_SHARED((r_agg, D), jnp.float32),
            pltpu.SemaphoreType.DMA,
            pltpu.SemaphoreType.DMA,
        ],
    )
    def scat_kernel(xt_hbm, a_hbm, ixt_hbm, idst_hbm, nb_hbm, out_hbm,
                    ixt_all, idst_all, a_all, idx2_v, g_xt, out_buf,
                    nb_v, agg, sem0, sem1):
        sid = lax.axis_index("s")
        sems = (sem0, sem1)
        tbase = sid * pw
        pc = pw // 8
        for q in range(8):
            pltpu.sync_copy(ixt_hbm.at[pl.ds(tbase + q * pc, pc)],
                            ixt_all.at[pl.ds(q * pc, pc)])
            pltpu.sync_copy(idst_hbm.at[pl.ds(tbase + q * pc, pc)],
                            idst_all.at[pl.ds(q * pc, pc)])
            pltpu.sync_copy(a_hbm.at[pl.ds(tbase + q * pc, pc)],
                            a_all.at[pl.ds(q * pc, pc)])
        pltpu.sync_copy(nb_hbm, nb_v)
        cid = lax.axis_index("c")
        nb16 = nb_v[...] + cid * half

        ob0 = out_buf.at[0]

        def zrow(i, _):
            for s in range(D // 16):
                ob0[i, pl.ds(s * 16, 16)] = jnp.zeros((16,), jnp.float32)
            return 0
        lax.fori_loop(0, SC_K, zrow, 0)
        done = 0
        while done < zrows:
            step = min(SC_K, zrows - done)
            pltpu.sync_copy(ob0.at[pl.ds(0, step)],
                            agg.at[pl.ds(sid * zrows + done, step)])
            done += step
        plsc.subcore_barrier()

        def issue(j, b):
            pltpu.async_copy(
                xt_hbm.at[ixt_all.at[pl.ds(j * SC_K, SC_K)]],
                g_xt.at[b], sems[b])

        issue(0, 0)
        issue(1, 1)

        def body(jj, _):
            for b in range(2):
                j = jj * 2 + b
                pltpu.make_async_copy(
                    xt_hbm.at[ixt_all.at[pl.ds(j * SC_K, SC_K)]],
                    g_xt.at[b], sems[b]).wait()
                for s in range(SC_K // 16):
                    d = idst_all[pl.ds(j * SC_K + s * 16, 16)]
                    inr = (d >= nb16) & (d < nb16 + half)
                    idx2_v.at[b][pl.ds(s * 16, 16)] = jnp.where(
                        inr, d - nb16, TRASH)
                gxb, obb = g_xt.at[b], out_buf.at[b]

                def group(g, _):
                    base_e = j * SC_K + g * 16
                    for ee in range(16):
                        e = g * 16 + ee
                        es = lax.broadcast_in_dim(base_e + ee, (16,), ())
                        av = plsc.load_gather(a_all, [es])
                        for s in range(D // 16):
                            sl = pl.ds(s * 16, 16)
                            obb[e, sl] = gxb[e, sl] * av
                    return 0
                lax.fori_loop(0, SC_K // 16, group, 0)
                @pl.when(j + 2 < nchunks)
                def _():
                    pltpu.async_copy(
                        xt_hbm.at[ixt_all.at[pl.ds((j + 2) * SC_K, SC_K)]],
                        g_xt.at[b], sems[b])
                pltpu.sync_copy(obb, agg.at[idx2_v.at[b]], add=True)
            return 0

        lax.fori_loop(0, nchunks // 2, body, 0)
        plsc.subcore_barrier()

        r0 = sid * drows
        pltpu.sync_copy(agg.at[pl.ds(r0, drows)],
                        out_hbm.at[cid, pl.ds(r0, drows)])

    return scat_kernel


def kernel(feat, edge_index, edge_type, edge_label, node_graph_ids, head_ids,
           tail_ids, rel_labels, weight_bases, w_comp, self_loop, A_w, A_b,
           B_w, B_b, attn_tab, rel_tab, fc_w, fc_b):
    N, D = feat.shape
    E = edge_index.shape[1]
    L, NB = weight_bases.shape[0], weight_bases.shape[1]
    R = w_comp.shape[1]
    AD = attn_tab.shape[1]
    RD = rel_tab.shape[1]
    B = head_ids.shape[0]

    npad = _ceil_to(N, max(NBLK, SC_NS * SC_K))
    epad = _ceil_to(E, 2 * SC_NS * SC_K)

    src = edge_index[0]
    dst = edge_index[1]
    pad_e = epad - E
    trash = jnp.int32(N)
    src_p = jnp.concatenate([src, jnp.zeros((pad_e,), jnp.int32)])
    dst_p = jnp.concatenate([dst, jnp.full((pad_e,), trash, jnp.int32)])
    ixt_base = edge_type * npad + src
    ixt_p = jnp.concatenate([ixt_base, jnp.zeros((pad_e,), jnp.int32)])
    it_base = edge_type * R + edge_label
    it_p = jnp.concatenate([it_base, jnp.zeros((pad_e,), jnp.int32)])

    h0 = jnp.pad(feat, ((0, npad - N), (0, 0)))

    a1 = A_w[:, :D, :]
    a2 = A_w[:, D:2 * D, :]
    a3 = A_w[:, 2 * D:2 * D + AD, :]
    a4 = A_w[:, 2 * D + AD:, :]

    bases_flat = weight_bases.reshape(L, NB, D * D)
    w_all, t_all = _prep_tables(w_comp, bases_flat, a3, a4,
                                A_b.reshape(L, 1, D), attn_tab,
                                L, R, NB, D, AD)
    w_all = w_all.reshape(L, R, D, D)
    t_all = t_all.reshape(L, R * R, D)

    attn_fn = _make_attn_kernel(epad, D)
    nrng = _ceil_to((npad + 3) // 4, 128)
    scat_fn = _make_scat_kernel(npad, epad, D, nrng)
    node_bases = [jnp.full((16,), k * nrng, jnp.int32) for k in (0, 2)]

    bw_all = B_w[:, :, 0]
    bb_all = jnp.broadcast_to(B_b[:, :1] / 16.0, (L, 16)).astype(jnp.float32)

    def layer_step(h, p):
        w_l, a1l, a2l, sll, t_l, bw_l, bb_l = p
        xt, ps, pd = _layer_proj(h, w_l, a1l, a2l, R, D, npad)
        xt_flat = xt.reshape(R * npad, D)
        a_e = attn_fn(ps, pd, t_l, src_p, dst_p, it_p, bw_l, bb_l)

        aggs = [scat_fn(xt_flat, a_e, ixt_p, dst_p, nb).reshape(-1, D)
                for nb in node_bases]
        agg = jnp.concatenate(aggs, axis=0)[:npad]
        h = _layer_update(h, sll, agg, D, npad)
        return h, h

    _, reprs = lax.scan(
        layer_step, h0,
        (w_all, a1, a2, self_loop, t_all, bw_all, bb_all))
    rep_flat = jnp.moveaxis(reprs, 0, 1).reshape(npad, L * D)
    ids_p = jnp.concatenate(
        [node_graph_ids, jnp.full((npad - N,), B, jnp.int32)])
    ids2d = ids_p.reshape(npad // NBLK, NBLK)
    fc_b2 = fc_b.reshape(1, 1)
    out = _readout(rep_flat, ids2d, rel_tab, fc_w, fc_b2,
                   head_ids, tail_ids, rel_labels, B, L * D, RD, npad)
    return out


# speedup vs baseline: 1.0622x; 1.0622x over previous
import functools

import jax
import jax.numpy as jnp
from jax import lax
from jax.experimental import pallas as pl
from jax.experimental.pallas import tpu as pltpu
from jax.experimental.pallas import tpu_sc as plsc

NBLK = 256
SC_K = 128
SC_NC = 1
SC_NS = 16


def _ceil_to(x, m):
    return (x + m - 1) // m * m


def _prep_body(wc_ref, wb_ref, a3_ref, a4_ref, ab_ref, attn_ref, w_ref, t_ref):
    w_ref[0] = jnp.dot(wc_ref[0], wb_ref[0], preferred_element_type=jnp.float32)
    pet = jnp.dot(attn_ref[...], a3_ref[0], preferred_element_type=jnp.float32)
    pel = jnp.dot(attn_ref[...], a4_ref[0], preferred_element_type=jnp.float32)
    t = pet[:, None, :] + pel[None, :, :] + ab_ref[...]
    t_ref[0] = t


def _prep_tables(w_comp, bases_flat, a3, a4, a_b, attn_tab, L, R, NB, D, AD):
    return pl.pallas_call(
        _prep_body,
        grid=(L,),
        in_specs=[
            pl.BlockSpec((1, R, NB), lambda l: (l, 0, 0)),
            pl.BlockSpec((1, NB, D * D), lambda l: (l, 0, 0)),
            pl.BlockSpec((1, AD, D), lambda l: (l, 0, 0)),
            pl.BlockSpec((1, AD, D), lambda l: (l, 0, 0)),
            pl.BlockSpec((1, 1, D), lambda l: (l, 0, 0)),
            pl.BlockSpec((R, AD), lambda l: (0, 0)),
        ],
        out_specs=[
            pl.BlockSpec((1, R, D * D), lambda l: (l, 0, 0)),
            pl.BlockSpec((1, R, R, D), lambda l: (l, 0, 0, 0)),
        ],
        out_shape=[
            jax.ShapeDtypeStruct((L, R, D * D), jnp.float32),
            jax.ShapeDtypeStruct((L, R, R, D), jnp.float32),
        ],
    )(w_comp, bases_flat, a3, a4, a_b, attn_tab)


def _proj_body(h_ref, w_ref, a1_ref, a2_ref, xt_ref, ps_ref, pd_ref, *, R):
    hb = h_ref[...]
    for r in range(R):
        xt_ref[r] = jnp.dot(hb, w_ref[r], preferred_element_type=jnp.float32)
    ps_ref[...] = jnp.dot(hb, a1_ref[...], preferred_element_type=jnp.float32)
    pd_ref[...] = jnp.dot(hb, a2_ref[...], preferred_element_type=jnp.float32)


def _layer_proj(h, w_l, a1, a2, R, D, npad):
    nb = npad // NBLK
    return pl.pallas_call(
        functools.partial(_proj_body, R=R),
        grid=(nb,),
        in_specs=[
            pl.BlockSpec((NBLK, D), lambda i: (i, 0)),
            pl.BlockSpec((R, D, D), lambda i: (0, 0, 0)),
            pl.BlockSpec((D, D), lambda i: (0, 0)),
            pl.BlockSpec((D, D), lambda i: (0, 0)),
        ],
        out_specs=[
            pl.BlockSpec((R, NBLK, D), lambda i: (0, i, 0)),
            pl.BlockSpec((NBLK, D), lambda i: (i, 0)),
            pl.BlockSpec((NBLK, D), lambda i: (i, 0)),
        ],
        out_shape=[
            jax.ShapeDtypeStruct((R, npad, D), jnp.float32),
            jax.ShapeDtypeStruct((npad, D), jnp.float32),
            jax.ShapeDtypeStruct((npad, D), jnp.float32),
        ],
    )(h, w_l, a1, a2)


def _update_body(h_ref, sl_ref, agg_ref, out_ref):
    cur = jnp.dot(h_ref[...], sl_ref[...], preferred_element_type=jnp.float32)
    out_ref[...] = jnp.maximum(cur + agg_ref[...], 0.0)


def _layer_update(h, sl_l, agg, D, npad):
    nb = npad // NBLK
    return pl.pallas_call(
        _update_body,
        grid=(nb,),
        in_specs=[
            pl.BlockSpec((NBLK, D), lambda i: (i, 0)),
            pl.BlockSpec((D, D), lambda i: (0, 0)),
            pl.BlockSpec((NBLK, D), lambda i: (i, 0)),
        ],
        out_specs=pl.BlockSpec((NBLK, D), lambda i: (i, 0)),
        out_shape=jax.ShapeDtypeStruct((npad, D), jnp.float32),
    )(h, sl_l, agg)


def _readout_body(rep_ref, ids_ref, rtab_ref, fcw_ref, fcb_ref,
                  head_ref, tail_ref, rl_ref, out_ref, *, nb, B, LD, RD):
    def blk(i, carry):
        acc, cnt = carry
        idsb = ids_ref[pl.ds(i, 1), :]
        oh = (lax.broadcasted_iota(jnp.int32, (B, NBLK), 0) == idsb)
        ohf = oh.astype(jnp.float32)
        chunk = rep_ref[pl.ds(i * NBLK, NBLK), :]
        acc = acc + jnp.dot(ohf, chunk, preferred_element_type=jnp.float32)
        cnt = cnt + jnp.sum(ohf, axis=1, keepdims=True)
        return acc, cnt

    acc0 = jnp.zeros((B, LD), jnp.float32)
    cnt0 = jnp.zeros((B, 1), jnp.float32)
    acc, cnt = lax.fori_loop(0, nb, blk, (acc0, cnt0))
    g_out = acc / jnp.maximum(cnt, 1.0)
    heads = jnp.concatenate(
        [rep_ref[pl.ds(head_ref[b], 1), :] for b in range(B)], axis=0)
    tails = jnp.concatenate(
        [rep_ref[pl.ds(tail_ref[b], 1), :] for b in range(B)], axis=0)
    rels = jnp.concatenate(
        [rtab_ref[pl.ds(rl_ref[b], 1), :] for b in range(B)], axis=0)
    g_rep = jnp.concatenate([g_out, heads, tails, rels], axis=1)
    out_ref[...] = (jnp.dot(g_rep, fcw_ref[...],
                            preferred_element_type=jnp.float32)
                    + fcb_ref[0, 0])


def _readout(rep_flat, ids2d, rel_tab, fc_w, fc_b2, head_ids, tail_ids,
             rel_labels, B, LD, RD, npad):
    nb = npad // NBLK
    return pl.pallas_call(
        functools.partial(_readout_body, nb=nb, B=B, LD=LD, RD=RD),
        in_specs=[
            pl.BlockSpec((npad, LD), lambda: (0, 0)),
            pl.BlockSpec((nb, NBLK), lambda: (0, 0)),
            pl.BlockSpec((B, RD), lambda: (0, 0)),
            pl.BlockSpec((LD * 3 + RD, 1), lambda: (0, 0)),
            pl.BlockSpec((1, 1), lambda: (0, 0)),
            pl.BlockSpec(memory_space=pltpu.SMEM),
            pl.BlockSpec(memory_space=pltpu.SMEM),
            pl.BlockSpec(memory_space=pltpu.SMEM),
        ],
        out_specs=pl.BlockSpec((B, 1), lambda: (0, 0)),
        out_shape=jax.ShapeDtypeStruct((B, 1), jnp.float32),
    )(rep_flat, ids2d, rel_tab, fc_w, fc_b2, head_ids, tail_ids, rel_labels)


def _make_attn_kernel(epad, D, R2T=256):
    nworkers = 2 * SC_NS
    pw = epad // nworkers
    nchunks = pw // SC_K
    assert nchunks % 2 == 0
    mesh = plsc.VectorSubcoreMesh(core_axis_name="c", subcore_axis_name="s",
                                  num_cores=2)

    @functools.partial(
        pl.kernel,
        out_type=jax.ShapeDtypeStruct((epad,), jnp.float32),
        mesh=mesh,
        compiler_params=pltpu.CompilerParams(needs_layout_passes=False),
        scratch_types=[
            pltpu.VMEM((pw,), jnp.int32),
            pltpu.VMEM((pw,), jnp.int32),
            pltpu.VMEM((pw,), jnp.int32),
            pltpu.VMEM((2, SC_K, D), jnp.float32),
            pltpu.VMEM((2, SC_K, D), jnp.float32),
            pltpu.VMEM((R2T, D), jnp.float32),
            pltpu.VMEM((2, SC_K), jnp.float32),
            pltpu.VMEM((D,), jnp.float32),
            pltpu.VMEM((16,), jnp.float32),
            pltpu.VMEM((256,), jnp.float32),
            pltpu.SemaphoreType.DMA,
            pltpu.SemaphoreType.DMA,
        ],
    )
    def attn_kernel(ps_hbm, pd_hbm, t_hbm, isrc_hbm, idst_hbm, it_hbm,
                    bw_hbm, bb_hbm, a_hbm,
                    isrc_all, idst_all, it_all, g_ps, g_pd, t_loc, a_buf,
                    bw_v, bb_v, acc_buf, sem0, sem1):
        cid = lax.axis_index("c")
        sid = lax.axis_index("s")
        wid = sid * 2 + cid
        tbase = wid * pw
        sems = (sem0, sem1)

        pc = pw // 8
        for q in range(8):
            pltpu.sync_copy(isrc_hbm.at[pl.ds(tbase + q * pc, pc)],
                            isrc_all.at[pl.ds(q * pc, pc)])
            pltpu.sync_copy(idst_hbm.at[pl.ds(tbase + q * pc, pc)],
                            idst_all.at[pl.ds(q * pc, pc)])
            pltpu.sync_copy(it_hbm.at[pl.ds(tbase + q * pc, pc)],
                            it_all.at[pl.ds(q * pc, pc)])
        pltpu.sync_copy(bw_hbm, bw_v)
        pltpu.sync_copy(bb_hbm, bb_v)
        bw_regs = [bw_v[pl.ds(s * 16, 16)] for s in range(D // 16)]
        bb16 = bb_v[...]
        lane16 = lax.broadcasted_iota(jnp.int32, (16,), 0) * 16

        def issue(j, b):
            sl = pl.ds(j * SC_K, SC_K)
            pltpu.async_copy(ps_hbm.at[isrc_all.at[sl]], g_ps.at[b], sems[b])
            pltpu.async_copy(pd_hbm.at[idst_all.at[sl]], g_pd.at[b], sems[b])

        def drain(j, b):
            sl = pl.ds(j * SC_K, SC_K)
            pltpu.make_async_copy(
                ps_hbm.at[isrc_all.at[sl]], g_ps.at[b], sems[b]).wait()
            pltpu.make_async_copy(
                pd_hbm.at[idst_all.at[sl]], g_pd.at[b], sems[b]).wait()

        for q in range(2):
            pltpu.sync_copy(t_hbm.at[pl.ds(q * R2T // 2, R2T // 2)],
                            t_loc.at[pl.ds(q * R2T // 2, R2T // 2)])
        lane = lax.broadcasted_iota(jnp.int32, (16,), 0)
        cols = [lane + s * 16 for s in range(D // 16)]
        issue(0, 0)
        issue(1, 1)

        def body(jj, _):
            for b in range(2):
                j = jj * 2 + b
                drain(j, b)
                psb, pdb = g_ps.at[b], g_pd.at[b]

                def group(g, _):
                    e0 = g * 16
                    for ee in range(16):
                        e = e0 + ee
                        esp = lax.broadcast_in_dim(
                            j * SC_K + e, (16,), ())
                        itv = plsc.load_gather(it_all, [esp])
                        acc = bb16
                        for s in range(D // 16):
                            sl = pl.ds(s * 16, 16)
                            tv = plsc.load_gather(t_loc, [itv, cols[s]])
                            pre = psb[e, sl] + pdb[e, sl] + tv
                            acc = acc + jnp.maximum(pre, 0.0) * bw_regs[s]
                        acc_buf[pl.ds(ee * 16, 16)] = acc
                    tots = jnp.zeros((16,), jnp.float32)
                    for s in range(16):
                        tots = tots + plsc.load_gather(acc_buf, [lane16 + s])
                    av = 1.0 / (1.0 + jnp.exp(-tots))
                    a_buf.at[b][pl.ds(e0, 16)] = av
                    return 0
                lax.fori_loop(0, SC_K // 16, group, 0)
                @pl.when(j + 2 < nchunks)
                def _():
                    issue_j = j + 2
                    sl = pl.ds(issue_j * SC_K, SC_K)
                    pltpu.async_copy(ps_hbm.at[isrc_all.at[sl]],
                                     g_ps.at[b], sems[b])
                    pltpu.async_copy(pd_hbm.at[idst_all.at[sl]],
                                     g_pd.at[b], sems[b])
                pltpu.sync_copy(a_buf.at[b],
                                a_hbm.at[pl.ds(tbase + j * SC_K, SC_K)])
            return 0

        lax.fori_loop(0, nchunks // 2, body, 0)

    return attn_kernel


def _make_scat_kernel(npad, epad, D, rng):
    half = rng
    TRASH = half
    r_agg = half + 128
    nworkers = SC_NS
    pw = epad // nworkers
    nchunks = pw // SC_K
    assert nchunks % 2 == 0
    zrows = r_agg // SC_NS
    drows = half // SC_NS
    mesh = plsc.VectorSubcoreMesh(core_axis_name="c", subcore_axis_name="s",
                                  num_cores=2)

    @functools.partial(
        pl.kernel,
        out_type=jax.ShapeDtypeStruct((2, half, D), jnp.float32),
        mesh=mesh,
        compiler_params=pltpu.CompilerParams(needs_layout_passes=False),
        scratch_types=[
            pltpu.VMEM((pw,), jnp.int32),
            pltpu.VMEM((pw,), jnp.int32),
            pltpu.VMEM((pw,), jnp.float32),
            pltpu.VMEM((2, SC_K), jnp.int32),
            pltpu.VMEM((2, SC_K, D), jnp.float32),
            pltpu.VMEM((2, SC_K, D), jnp.float32),
            pltpu.VMEM((16,), jnp.int32),
            pltpu.VMEM_SHARED((r_agg, D), jnp.float32),
            pltpu.SemaphoreType.DMA,
            pltpu.SemaphoreType.DMA,
        ],
    )
    def scat_kernel(xt_hbm, a_hbm, ixt_hbm, idst_hbm, nb_hbm, out_hbm,
                    ixt_all, idst_all, a_all, idx2_v, g_xt, out_buf,
                    nb_v, agg, sem0, sem1):
        sid = lax.axis_index("s")
        sems = (sem0, sem1)
        tbase = sid * pw
        pc = pw // 8
        for q in range(8):
            pltpu.sync_copy(ixt_hbm.at[pl.ds(tbase + q * pc, pc)],
                            ixt_all.at[pl.ds(q * pc, pc)])
            pltpu.sync_copy(idst_hbm.at[pl.ds(tbase + q * pc, pc)],
                            idst_all.at[pl.ds(q * pc, pc)])
            pltpu.sync_copy(a_hbm.at[pl.ds(tbase + q * pc, pc)],
                            a_all.at[pl.ds(q * pc, pc)])
        pltpu.sync_copy(nb_hbm, nb_v)
        cid = lax.axis_index("c")
        nb16 = nb_v[...] + cid * half

        ob0 = out_buf.at[0]

        def zrow(i, _):
            for s in range(D // 16):
                ob0[i, pl.ds(s * 16, 16)] = jnp.zeros((16,), jnp.float32)
            return 0
        lax.fori_loop(0, SC_K, zrow, 0)
        done = 0
        while done < zrows:
            step = min(SC_K, zrows - done)
            pltpu.sync_copy(ob0.at[pl.ds(0, step)],
                            agg.at[pl.ds(sid * zrows + done, step)])
            done += step
        plsc.subcore_barrier()

        def issue(j, b):
            pltpu.async_copy(
                xt_hbm.at[ixt_all.at[pl.ds(j * SC_K, SC_K)]],
                g_xt.at[b], sems[b])

        issue(0, 0)
        issue(1, 1)

        def body(jj, _):
            for b in range(2):
                j = jj * 2 + b
                pltpu.make_async_copy(
                    xt_hbm.at[ixt_all.at[pl.ds(j * SC_K, SC_K)]],
                    g_xt.at[b], sems[b]).wait()
                for s in range(SC_K // 16):
                    d = idst_all[pl.ds(j * SC_K + s * 16, 16)]
                    inr = (d >= nb16) & (d < nb16 + half)
                    idx2_v.at[b][pl.ds(s * 16, 16)] = jnp.where(
                        inr, d - nb16, TRASH)
                gxb, obb = g_xt.at[b], out_buf.at[b]

                def group(g, _):
                    base_e = j * SC_K + g * 16
                    for ee in range(16):
                        e = g * 16 + ee
                        es = lax.broadcast_in_dim(base_e + ee, (16,), ())
                        av = plsc.load_gather(a_all, [es])
                        for s in range(D // 16):
                            sl = pl.ds(s * 16, 16)
                            obb[e, sl] = gxb[e, sl] * av
                    return 0
                lax.fori_loop(0, SC_K // 16, group, 0)
                @pl.when(j + 2 < nchunks)
                def _():
                    pltpu.async_copy(
                        xt_hbm.at[ixt_all.at[pl.ds((j + 2) * SC_K, SC_K)]],
                        g_xt.at[b], sems[b])
                pltpu.sync_copy(obb, agg.at[idx2_v.at[b]], add=True)
            return 0

        lax.fori_loop(0, nchunks // 2, body, 0)
        plsc.subcore_barrier()

        r0 = sid * drows
        pltpu.sync_copy(agg.at[pl.ds(r0, drows)],
                        out_hbm.at[cid, pl.ds(r0, drows)])

    return scat_kernel


def kernel(feat, edge_index, edge_type, edge_label, node_graph_ids, head_ids,
           tail_ids, rel_labels, weight_bases, w_comp, self_loop, A_w, A_b,
           B_w, B_b, attn_tab, rel_tab, fc_w, fc_b):
    N, D = feat.shape
    E = edge_index.shape[1]
    L, NB = weight_bases.shape[0], weight_bases.shape[1]
    R = w_comp.shape[1]
    AD = attn_tab.shape[1]
    RD = rel_tab.shape[1]
    B = head_ids.shape[0]

    npad = _ceil_to(N, max(NBLK, SC_NS * SC_K))
    epad = _ceil_to(E, 2 * SC_NS * SC_K)

    src = edge_index[0]
    dst = edge_index[1]
    pad_e = epad - E
    trash = jnp.int32(N)
    src_p = jnp.concatenate([src, jnp.zeros((pad_e,), jnp.int32)])
    dst_p = jnp.concatenate([dst, jnp.full((pad_e,), trash, jnp.int32)])
    ixt_base = edge_type * npad + src
    ixt_p = jnp.concatenate([ixt_base, jnp.zeros((pad_e,), jnp.int32)])
    it_base = edge_type * R + edge_label
    it_p = jnp.concatenate([it_base, jnp.zeros((pad_e,), jnp.int32)])

    h0 = jnp.pad(feat, ((0, npad - N), (0, 0)))

    a1 = A_w[:, :D, :]
    a2 = A_w[:, D:2 * D, :]
    a3 = A_w[:, 2 * D:2 * D + AD, :]
    a4 = A_w[:, 2 * D + AD:, :]

    bases_flat = weight_bases.reshape(L, NB, D * D)
    w_all, t_all = _prep_tables(w_comp, bases_flat, a3, a4,
                                A_b.reshape(L, 1, D), attn_tab,
                                L, R, NB, D, AD)
    w_all = w_all.reshape(L, R, D, D)
    t_all = t_all.reshape(L, R * R, D)

    attn_fn = _make_attn_kernel(epad, D)
    nrng = _ceil_to((npad + 3) // 4, 128)
    scat_fn = _make_scat_kernel(npad, epad, D, nrng)
    node_bases = [jnp.full((16,), k * nrng, jnp.int32) for k in (0, 2)]

    bw_all = B_w[:, :, 0]
    bb_all = jnp.broadcast_to(B_b[:, :1] / 16.0, (L, 16)).astype(jnp.float32)

    def layer_step(h, p):
        w_l, a1l, a2l, sll, t_l, bw_l, bb_l = p
        xt, ps, pd = _layer_proj(h, w_l, a1l, a2l, R, D, npad)
        xt_flat = xt.reshape(R * npad, D)
        a_e = attn_fn(ps, pd, t_l, src_p, dst_p, it_p, bw_l, bb_l)

        aggs = [scat_fn(xt_flat, a_e, ixt_p, dst_p, nb).reshape(-1, D)
                for nb in node_bases]
        agg = jnp.concatenate(aggs, axis=0)[:npad]
        h = _layer_update(h, sll, agg, D, npad)
        return h, h

    _, reprs = lax.scan(
        layer_step, h0,
        (w_all, a1, a2, self_loop, t_all, bw_all, bb_all))
    rep_flat = jnp.moveaxis(reprs, 0, 1).reshape(npad, L * D)
    ids_p = jnp.concatenate(
        [node_graph_ids, jnp.full((npad - N,), B, jnp.int32)])
    ids2d = ids_p.reshape(npad // NBLK, NBLK)
    fc_b2 = fc_b.reshape(1, 1)
    out = _readout(rep_flat, ids2d, rel_tab, fc_w, fc_b2,
                   head_ids, tail_ids, rel_labels, B, L * D, RD, npad)
    return out


# speedup vs baseline: 1.0770x; 1.0139x over previous
import functools

import jax
import jax.numpy as jnp
from jax import lax
from jax.experimental import pallas as pl
from jax.experimental.pallas import tpu as pltpu
from jax.experimental.pallas import tpu_sc as plsc

NBLK = 256
SC_K = 128
SC_NC = 1
SC_NS = 16


def _ceil_to(x, m):
    return (x + m - 1) // m * m


def _prep_body(wc_ref, wb_ref, a3_ref, a4_ref, ab_ref, attn_ref, w_ref, t_ref):
    w_ref[0] = jnp.dot(wc_ref[0], wb_ref[0], preferred_element_type=jnp.float32)
    pet = jnp.dot(attn_ref[...], a3_ref[0], preferred_element_type=jnp.float32)
    pel = jnp.dot(attn_ref[...], a4_ref[0], preferred_element_type=jnp.float32)
    t = pet[:, None, :] + pel[None, :, :] + ab_ref[...]
    t_ref[0] = t


def _prep_tables(w_comp, bases_flat, a3, a4, a_b, attn_tab, L, R, NB, D, AD):
    return pl.pallas_call(
        _prep_body,
        grid=(L,),
        in_specs=[
            pl.BlockSpec((1, R, NB), lambda l: (l, 0, 0)),
            pl.BlockSpec((1, NB, D * D), lambda l: (l, 0, 0)),
            pl.BlockSpec((1, AD, D), lambda l: (l, 0, 0)),
            pl.BlockSpec((1, AD, D), lambda l: (l, 0, 0)),
            pl.BlockSpec((1, 1, D), lambda l: (l, 0, 0)),
            pl.BlockSpec((R, AD), lambda l: (0, 0)),
        ],
        out_specs=[
            pl.BlockSpec((1, R, D * D), lambda l: (l, 0, 0)),
            pl.BlockSpec((1, R, R, D), lambda l: (l, 0, 0, 0)),
        ],
        out_shape=[
            jax.ShapeDtypeStruct((L, R, D * D), jnp.float32),
            jax.ShapeDtypeStruct((L, R, R, D), jnp.float32),
        ],
    )(w_comp, bases_flat, a3, a4, a_b, attn_tab)


def _proj_body(h_ref, w_ref, a1_ref, a2_ref, xt_ref, ps_ref, pd_ref, *, R):
    hb = h_ref[...]
    for r in range(R):
        xt_ref[r] = jnp.dot(hb, w_ref[r], preferred_element_type=jnp.float32)
    ps_ref[...] = jnp.dot(hb, a1_ref[...], preferred_element_type=jnp.float32)
    pd_ref[...] = jnp.dot(hb, a2_ref[...], preferred_element_type=jnp.float32)


def _layer_proj(h, w_l, a1, a2, R, D, npad):
    nb = npad // NBLK
    return pl.pallas_call(
        functools.partial(_proj_body, R=R),
        grid=(nb,),
        in_specs=[
            pl.BlockSpec((NBLK, D), lambda i: (i, 0)),
            pl.BlockSpec((R, D, D), lambda i: (0, 0, 0)),
            pl.BlockSpec((D, D), lambda i: (0, 0)),
            pl.BlockSpec((D, D), lambda i: (0, 0)),
        ],
        out_specs=[
            pl.BlockSpec((R, NBLK, D), lambda i: (0, i, 0)),
            pl.BlockSpec((NBLK, D), lambda i: (i, 0)),
            pl.BlockSpec((NBLK, D), lambda i: (i, 0)),
        ],
        out_shape=[
            jax.ShapeDtypeStruct((R, npad, D), jnp.float32),
            jax.ShapeDtypeStruct((npad, D), jnp.float32),
            jax.ShapeDtypeStruct((npad, D), jnp.float32),
        ],
    )(h, w_l, a1, a2)


def _update_body(h_ref, sl_ref, agg_ref, out_ref):
    cur = jnp.dot(h_ref[...], sl_ref[...], preferred_element_type=jnp.float32)
    out_ref[...] = jnp.maximum(cur + agg_ref[...], 0.0)


def _layer_update(h, sl_l, agg, D, npad):
    nb = npad // NBLK
    return pl.pallas_call(
        _update_body,
        grid=(nb,),
        in_specs=[
            pl.BlockSpec((NBLK, D), lambda i: (i, 0)),
            pl.BlockSpec((D, D), lambda i: (0, 0)),
            pl.BlockSpec((NBLK, D), lambda i: (i, 0)),
        ],
        out_specs=pl.BlockSpec((NBLK, D), lambda i: (i, 0)),
        out_shape=jax.ShapeDtypeStruct((npad, D), jnp.float32),
    )(h, sl_l, agg)


def _readout_body(rep_ref, ids_ref, rtab_ref, fcw_ref, fcb_ref,
                  head_ref, tail_ref, rl_ref, out_ref, *, nb, B, LD, RD):
    def blk(i, carry):
        acc, cnt = carry
        idsb = ids_ref[pl.ds(i, 1), :]
        oh = (lax.broadcasted_iota(jnp.int32, (B, NBLK), 0) == idsb)
        ohf = oh.astype(jnp.float32)
        chunk = rep_ref[pl.ds(i * NBLK, NBLK), :]
        acc = acc + jnp.dot(ohf, chunk, preferred_element_type=jnp.float32)
        cnt = cnt + jnp.sum(ohf, axis=1, keepdims=True)
        return acc, cnt

    acc0 = jnp.zeros((B, LD), jnp.float32)
    cnt0 = jnp.zeros((B, 1), jnp.float32)
    acc, cnt = lax.fori_loop(0, nb, blk, (acc0, cnt0))
    g_out = acc / jnp.maximum(cnt, 1.0)
    heads = jnp.concatenate(
        [rep_ref[pl.ds(head_ref[b], 1), :] for b in range(B)], axis=0)
    tails = jnp.concatenate(
        [rep_ref[pl.ds(tail_ref[b], 1), :] for b in range(B)], axis=0)
    rels = jnp.concatenate(
        [rtab_ref[pl.ds(rl_ref[b], 1), :] for b in range(B)], axis=0)
    g_rep = jnp.concatenate([g_out, heads, tails, rels], axis=1)
    out_ref[...] = (jnp.dot(g_rep, fcw_ref[...],
                            preferred_element_type=jnp.float32)
                    + fcb_ref[0, 0])


def _readout(rep_flat, ids2d, rel_tab, fc_w, fc_b2, head_ids, tail_ids,
             rel_labels, B, LD, RD, npad):
    nb = npad // NBLK
    return pl.pallas_call(
        functools.partial(_readout_body, nb=nb, B=B, LD=LD, RD=RD),
        in_specs=[
            pl.BlockSpec((npad, LD), lambda: (0, 0)),
            pl.BlockSpec((nb, NBLK), lambda: (0, 0)),
            pl.BlockSpec((B, RD), lambda: (0, 0)),
            pl.BlockSpec((LD * 3 + RD, 1), lambda: (0, 0)),
            pl.BlockSpec((1, 1), lambda: (0, 0)),
            pl.BlockSpec(memory_space=pltpu.SMEM),
            pl.BlockSpec(memory_space=pltpu.SMEM),
            pl.BlockSpec(memory_space=pltpu.SMEM),
        ],
        out_specs=pl.BlockSpec((B, 1), lambda: (0, 0)),
        out_shape=jax.ShapeDtypeStruct((B, 1), jnp.float32),
    )(rep_flat, ids2d, rel_tab, fc_w, fc_b2, head_ids, tail_ids, rel_labels)


def _make_attn_kernel(epad, D, R2T=256):
    nworkers = 2 * SC_NS
    pw = epad // nworkers
    nchunks = pw // SC_K
    assert nchunks % 2 == 0
    mesh = plsc.VectorSubcoreMesh(core_axis_name="c", subcore_axis_name="s",
                                  num_cores=2)

    @functools.partial(
        pl.kernel,
        out_type=jax.ShapeDtypeStruct((epad,), jnp.float32),
        mesh=mesh,
        compiler_params=pltpu.CompilerParams(needs_layout_passes=False),
        scratch_types=[
            pltpu.VMEM((pw,), jnp.int32),
            pltpu.VMEM((pw,), jnp.int32),
            pltpu.VMEM((pw,), jnp.int32),
            pltpu.VMEM((2, SC_K, D), jnp.float32),
            pltpu.VMEM((2, SC_K, D), jnp.float32),
            pltpu.VMEM((R2T, D), jnp.float32),
            pltpu.VMEM((2, SC_K), jnp.float32),
            pltpu.VMEM((D,), jnp.float32),
            pltpu.VMEM((16,), jnp.float32),
            pltpu.VMEM((256,), jnp.float32),
            pltpu.SemaphoreType.DMA,
            pltpu.SemaphoreType.DMA,
        ],
    )
    def attn_kernel(ps_hbm, pd_hbm, t_hbm, isrc_hbm, idst_hbm, it_hbm,
                    bw_hbm, bb_hbm, a_hbm,
                    isrc_all, idst_all, it_all, g_ps, g_pd, t_loc, a_buf,
                    bw_v, bb_v, acc_buf, sem0, sem1):
        cid = lax.axis_index("c")
        sid = lax.axis_index("s")
        wid = sid * 2 + cid
        tbase = wid * pw
        sems = (sem0, sem1)

        pc = pw // 8
        for q in range(8):
            pltpu.sync_copy(isrc_hbm.at[pl.ds(tbase + q * pc, pc)],
                            isrc_all.at[pl.ds(q * pc, pc)])
            pltpu.sync_copy(idst_hbm.at[pl.ds(tbase + q * pc, pc)],
                            idst_all.at[pl.ds(q * pc, pc)])
            pltpu.sync_copy(it_hbm.at[pl.ds(tbase + q * pc, pc)],
                            it_all.at[pl.ds(q * pc, pc)])
        pltpu.sync_copy(bw_hbm, bw_v)
        pltpu.sync_copy(bb_hbm, bb_v)
        bw_regs = [bw_v[pl.ds(s * 16, 16)] for s in range(D // 16)]
        bb16 = bb_v[...]
        lane16 = lax.broadcasted_iota(jnp.int32, (16,), 0) * 16

        def issue(j, b):
            sl = pl.ds(j * SC_K, SC_K)
            pltpu.async_copy(ps_hbm.at[isrc_all.at[sl]], g_ps.at[b], sems[b])
            pltpu.async_copy(pd_hbm.at[idst_all.at[sl]], g_pd.at[b], sems[b])

        def drain(j, b):
            sl = pl.ds(j * SC_K, SC_K)
            pltpu.make_async_copy(
                ps_hbm.at[isrc_all.at[sl]], g_ps.at[b], sems[b]).wait()
            pltpu.make_async_copy(
                pd_hbm.at[idst_all.at[sl]], g_pd.at[b], sems[b]).wait()

        for q in range(2):
            pltpu.sync_copy(t_hbm.at[pl.ds(q * R2T // 2, R2T // 2)],
                            t_loc.at[pl.ds(q * R2T // 2, R2T // 2)])
        lane = lax.broadcasted_iota(jnp.int32, (16,), 0)
        cols = [lane + s * 16 for s in range(D // 16)]
        issue(0, 0)
        issue(1, 1)

        def body(jj, _):
            for b in range(2):
                j = jj * 2 + b
                drain(j, b)
                psb, pdb = g_ps.at[b], g_pd.at[b]

                def group(g, _):
                    e0 = g * 16
                    for ee in range(16):
                        e = e0 + ee
                        esp = lax.broadcast_in_dim(
                            j * SC_K + e, (16,), ())
                        itv = plsc.load_gather(it_all, [esp])
                        acc = bb16
                        for s in range(D // 16):
                            sl = pl.ds(s * 16, 16)
                            tv = plsc.load_gather(t_loc, [itv, cols[s]])
                            pre = psb[e, sl] + pdb[e, sl] + tv
                            acc = acc + jnp.maximum(pre, 0.0) * bw_regs[s]
                        acc_buf[pl.ds(ee * 16, 16)] = acc
                    tots = jnp.zeros((16,), jnp.float32)
                    for s in range(16):
                        tots = tots + plsc.load_gather(acc_buf, [lane16 + s])
                    av = 1.0 / (1.0 + jnp.exp(-tots))
                    a_buf.at[b][pl.ds(e0, 16)] = av
                    return 0
                lax.fori_loop(0, SC_K // 16, group, 0)
                @pl.when(j + 2 < nchunks)
                def _():
                    issue_j = j + 2
                    sl = pl.ds(issue_j * SC_K, SC_K)
                    pltpu.async_copy(ps_hbm.at[isrc_all.at[sl]],
                                     g_ps.at[b], sems[b])
                    pltpu.async_copy(pd_hbm.at[idst_all.at[sl]],
                                     g_pd.at[b], sems[b])
                pltpu.sync_copy(a_buf.at[b],
                                a_hbm.at[pl.ds(tbase + j * SC_K, SC_K)])
            return 0

        lax.fori_loop(0, nchunks // 2, body, 0)

    return attn_kernel


def _make_scat_kernel(npad, epad, D, rng):
    half = rng
    TRASH = half
    r_agg = half + 128
    nworkers = SC_NS
    pw = epad // nworkers
    nchunks = pw // SC_K
    assert nchunks % 2 == 0
    zrows = r_agg // SC_NS
    drows = half // SC_NS
    mesh = plsc.VectorSubcoreMesh(core_axis_name="c", subcore_axis_name="s",
                                  num_cores=1)

    @functools.partial(
        pl.kernel,
        out_type=jax.ShapeDtypeStruct((half, D), jnp.float32),
        mesh=mesh,
        compiler_params=pltpu.CompilerParams(needs_layout_passes=False),
        scratch_types=[
            pltpu.VMEM((pw,), jnp.int32),
            pltpu.VMEM((pw,), jnp.int32),
            pltpu.VMEM((pw,), jnp.float32),
            pltpu.VMEM((2, SC_K), jnp.int32),
            pltpu.VMEM((2, SC_K, D), jnp.float32),
            pltpu.VMEM((2, SC_K, D), jnp.float32),
            pltpu.VMEM((16,), jnp.int32),
            pltpu.VMEM_SHARED((r_agg, D), jnp.float32),
            pltpu.SemaphoreType.DMA,
            pltpu.SemaphoreType.DMA,
        ],
    )
    def scat_kernel(xt_hbm, a_hbm, ixt_hbm, idst_hbm, nb_hbm, out_hbm,
                    ixt_all, idst_all, a_all, idx2_v, g_xt, out_buf,
                    nb_v, agg, sem0, sem1):
        sid = lax.axis_index("s")
        sems = (sem0, sem1)
        tbase = sid * pw
        pc = pw // 8
        for q in range(8):
            pltpu.sync_copy(ixt_hbm.at[pl.ds(tbase + q * pc, pc)],
                            ixt_all.at[pl.ds(q * pc, pc)])
            pltpu.sync_copy(idst_hbm.at[pl.ds(tbase + q * pc, pc)],
                            idst_all.at[pl.ds(q * pc, pc)])
            pltpu.sync_copy(a_hbm.at[pl.ds(tbase + q * pc, pc)],
                            a_all.at[pl.ds(q * pc, pc)])
        pltpu.sync_copy(nb_hbm, nb_v)
        nb16 = nb_v[...]

        ob0 = out_buf.at[0]

        def zrow(i, _):
            for s in range(D // 16):
                ob0[i, pl.ds(s * 16, 16)] = jnp.zeros((16,), jnp.float32)
            return 0
        lax.fori_loop(0, SC_K, zrow, 0)
        done = 0
        while done < zrows:
            step = min(SC_K, zrows - done)
            pltpu.sync_copy(ob0.at[pl.ds(0, step)],
                            agg.at[pl.ds(sid * zrows + done, step)])
            done += step
        plsc.subcore_barrier()

        def issue(j, b):
            pltpu.async_copy(
                xt_hbm.at[ixt_all.at[pl.ds(j * SC_K, SC_K)]],
                g_xt.at[b], sems[b])

        issue(0, 0)
        issue(1, 1)

        def body(jj, _):
            for b in range(2):
                j = jj * 2 + b
                pltpu.make_async_copy(
                    xt_hbm.at[ixt_all.at[pl.ds(j * SC_K, SC_K)]],
                    g_xt.at[b], sems[b]).wait()
                for s in range(SC_K // 16):
                    d = idst_all[pl.ds(j * SC_K + s * 16, 16)]
                    inr = (d >= nb16) & (d < nb16 + half)
                    idx2_v.at[b][pl.ds(s * 16, 16)] = jnp.where(
                        inr, d - nb16, TRASH)
                gxb, obb = g_xt.at[b], out_buf.at[b]

                def group(g, _):
                    base_e = j * SC_K + g * 16
                    for ee in range(16):
                        e = g * 16 + ee
                        es = lax.broadcast_in_dim(base_e + ee, (16,), ())
                        av = plsc.load_gather(a_all, [es])
                        for s in range(D // 16):
                            sl = pl.ds(s * 16, 16)
                            obb[e, sl] = gxb[e, sl] * av
                    return 0
                lax.fori_loop(0, SC_K // 16, group, 0)
                @pl.when(j + 2 < nchunks)
                def _():
                    pltpu.async_copy(
                        xt_hbm.at[ixt_all.at[pl.ds((j + 2) * SC_K, SC_K)]],
                        g_xt.at[b], sems[b])
                pltpu.sync_copy(obb, agg.at[idx2_v.at[b]], add=True)
            return 0

        lax.fori_loop(0, nchunks // 2, body, 0)
        plsc.subcore_barrier()

        r0 = sid * drows
        pltpu.sync_copy(agg.at[pl.ds(r0, drows)],
                        out_hbm.at[pl.ds(r0, drows)])

    return scat_kernel


def kernel(feat, edge_index, edge_type, edge_label, node_graph_ids, head_ids,
           tail_ids, rel_labels, weight_bases, w_comp, self_loop, A_w, A_b,
           B_w, B_b, attn_tab, rel_tab, fc_w, fc_b):
    N, D = feat.shape
    E = edge_index.shape[1]
    L, NB = weight_bases.shape[0], weight_bases.shape[1]
    R = w_comp.shape[1]
    AD = attn_tab.shape[1]
    RD = rel_tab.shape[1]
    B = head_ids.shape[0]

    npad = _ceil_to(N, max(NBLK, SC_NS * SC_K))
    epad = _ceil_to(E, 2 * SC_NS * SC_K)

    src = edge_index[0]
    dst = edge_index[1]
    pad_e = epad - E
    trash = jnp.int32(N)
    src_p = jnp.concatenate([src, jnp.zeros((pad_e,), jnp.int32)])
    dst_p = jnp.concatenate([dst, jnp.full((pad_e,), trash, jnp.int32)])
    ixt_base = edge_type * npad + src
    ixt_p = jnp.concatenate([ixt_base, jnp.zeros((pad_e,), jnp.int32)])
    it_base = edge_type * R + edge_label
    it_p = jnp.concatenate([it_base, jnp.zeros((pad_e,), jnp.int32)])

    h0 = jnp.pad(feat, ((0, npad - N), (0, 0)))

    a1 = A_w[:, :D, :]
    a2 = A_w[:, D:2 * D, :]
    a3 = A_w[:, 2 * D:2 * D + AD, :]
    a4 = A_w[:, 2 * D + AD:, :]

    bases_flat = weight_bases.reshape(L, NB, D * D)
    w_all, t_all = _prep_tables(w_comp, bases_flat, a3, a4,
                                A_b.reshape(L, 1, D), attn_tab,
                                L, R, NB, D, AD)
    w_all = w_all.reshape(L, R, D, D)
    t_all = t_all.reshape(L, R * R, D)

    attn_fn = _make_attn_kernel(epad, D)
    nrng = _ceil_to((npad + 2) // 3, 128)
    scat_fn = _make_scat_kernel(npad, epad, D, nrng)
    node_bases = [jnp.full((16,), k * nrng, jnp.int32) for k in range(3)]

    bw_all = B_w[:, :, 0]
    bb_all = jnp.broadcast_to(B_b[:, :1] / 16.0, (L, 16)).astype(jnp.float32)

    def layer_step(h, p):
        w_l, a1l, a2l, sll, t_l, bw_l, bb_l = p
        xt, ps, pd = _layer_proj(h, w_l, a1l, a2l, R, D, npad)
        xt_flat = xt.reshape(R * npad, D)
        a_e = attn_fn(ps, pd, t_l, src_p, dst_p, it_p, bw_l, bb_l)

        aggs = [scat_fn(xt_flat, a_e, ixt_p, dst_p, nb)
                for nb in node_bases]
        agg = jnp.concatenate(aggs, axis=0)[:npad]
        h = _layer_update(h, sll, agg, D, npad)
        return h, h

    _, reprs = lax.scan(
        layer_step, h0,
        (w_all, a1, a2, self_loop, t_all, bw_all, bb_all))
    rep_flat = jnp.moveaxis(reprs, 0, 1).reshape(npad, L * D)
    ids_p = jnp.concatenate(
        [node_graph_ids, jnp.full((npad - N,), B, jnp.int32)])
    ids2d = ids_p.reshape(npad // NBLK, NBLK)
    fc_b2 = fc_b.reshape(1, 1)
    out = _readout(rep_flat, ids2d, rel_tab, fc_w, fc_b2,
                   head_ids, tail_ids, rel_labels, B, L * D, RD, npad)
    return out
